# baseline (speedup 1.0000x reference)
import functools

import jax
import jax.numpy as jnp
from jax.experimental import pallas as pl
from jax.experimental.pallas import tpu as pltpu
from jax.experimental.pallas import tpu_sc as plsc

_SMESH = plsc.ScalarSubcoreMesh(axis_name="c", num_cores=1)


@functools.partial(
    pl.kernel,
    out_type=jax.ShapeDtypeStruct((1,), jnp.float32),
    mesh=_SMESH,
    compiler_params=pltpu.CompilerParams(needs_layout_passes=False),
    scratch_types=[
        pltpu.SMEM((42,), jnp.float32),
        pltpu.SMEM((7,), jnp.float32),
        pltpu.SMEM((1,), jnp.int32),
        pltpu.SMEM((1,), jnp.float32),
        pltpu.SemaphoreType.DMA,
        pltpu.SemaphoreType.DMA,
        pltpu.SemaphoreType.DMA,
    ],
)
def _scs_row_dot(m_hbm, t_hbm, s_hbm, out_hbm, m_s, t_s, s_s, o_s,
                 sem_m, sem_t, sem_s):
    cp_m = pltpu.async_copy(m_hbm, m_s, sem_m)
    cp_t = pltpu.async_copy(t_hbm, t_s, sem_t)
    cp_s = pltpu.async_copy(s_hbm, s_s, sem_s)
    cp_m.wait()
    cp_t.wait()
    cp_s.wait()
    base = s_s[0] * 7
    acc = m_s[base] * t_s[0]
    for j in range(1, 7):
        acc = acc + m_s[base + j] * t_s[j]
    o_s[0] = acc
    pltpu.sync_copy(o_s, out_hbm)


def kernel(state, M, theta):
    s_arr = jnp.asarray(state, jnp.int32).reshape(1)
    out = _scs_row_dot(M.reshape(42), theta, s_arr)
    return out.reshape(())
